# baseline (speedup 1.0000x reference)
import functools

import jax
import jax.numpy as jnp
from jax.experimental import pallas as pl
from jax.experimental.pallas import tpu as pltpu

B, T, C = 1, 2048, 768
H, D = 12, 64
K, P = 4, 2
ALPHA, BETA = 2.0, 32.0

TQ = 256
VP = 128


def _proj_kernel(x_ref, wqT_ref, wkT_ref, wvT_ref, wdqT_ref, wdkT_ref,
                 dq_ref, dkT_ref, v_ref):
    x = x_ref[...]
    qh = jnp.dot(x, wqT_ref[0], preferred_element_type=jnp.float32)
    kh = jnp.dot(x, wkT_ref[0], preferred_element_type=jnp.float32)
    scale = jnp.float32(P - 1)
    dq = jax.nn.sigmoid(
        jnp.dot(qh, wdqT_ref[...], preferred_element_type=jnp.float32)) * scale
    dk = jax.nn.sigmoid(
        jnp.dot(kh, wdkT_ref[...], preferred_element_type=jnp.float32)) * scale
    beta = jnp.float32(BETA)
    c0 = jnp.exp(jnp.float32(-BETA / 2))
    dq_ref[0] = jnp.concatenate(
        [jnp.exp(beta * dq), c0 * jnp.exp(-beta * dq)], axis=1)
    fkT = jnp.concatenate(
        [jnp.exp(beta * dk), c0 * jnp.exp(-beta * dk)], axis=1)
    dkT_ref[0] = fkT.T
    vh = jnp.dot(x, wvT_ref[0], preferred_element_type=jnp.float32)
    v_ref[0] = jnp.concatenate(
        [vh, jnp.ones((T, 1), jnp.float32), jnp.zeros((T, VP - D - 1), jnp.float32)],
        axis=1)


def _lcp_weights(fq, fkT):
    one = jnp.float32(1.0)
    two = jnp.float32(2.0)

    def level(l):
        a = fq[:, l:l + 1]
        ia = fq[:, K + l:K + l + 1]
        b = fkT[l:l + 1, :]
        ib = fkT[K + l:K + l + 1, :]
        return jnp.maximum(a * ib, ia * b)

    u0 = level(0)
    u1 = level(1)
    p1 = (one + u0) * (one + u1)
    lcp = (two + u1) / p1
    u2 = level(2)
    u3 = level(3)
    p3 = p1 * ((one + u2) * (one + u3))
    lcp = lcp + (two + u3) / p3
    return jnp.exp2(lcp)


def _attn_kernel(dq_ref, dkT_ref, v_ref, woT_ref, y_ref):
    i = pl.program_id(0)
    h = pl.program_id(1)
    dq = dq_ref[0]

    def body(j, acc):
        dkT = dkT_ref[0, :, pl.ds(j * TQ, TQ)]
        vblk = v_ref[0, pl.ds(j * TQ, TQ), :]
        w = _lcp_weights(dq, dkT)
        return acc + jnp.dot(w, vblk, preferred_element_type=jnp.float32)

    acc0 = jnp.zeros((TQ, VP), jnp.float32)
    acc = jax.lax.fori_loop(0, i, body, acc0)

    dkT = dkT_ref[0, :, pl.ds(i * TQ, TQ)]
    vblk = v_ref[0, pl.ds(i * TQ, TQ), :]
    w = _lcp_weights(dq, dkT)
    rows = jax.lax.broadcasted_iota(jnp.int32, (TQ, TQ), 0)
    cols = jax.lax.broadcasted_iota(jnp.int32, (TQ, TQ), 1)
    w = jnp.where(cols <= rows, w, jnp.float32(0.0))
    acc = acc + jnp.dot(w, vblk, preferred_element_type=jnp.float32)

    out = acc[:, :D] / acc[:, D:D + 1]
    y = jnp.dot(out, woT_ref[...], preferred_element_type=jnp.float32)

    @pl.when(h == 0)
    def _():
        y_ref[...] = y

    @pl.when(h > 0)
    def _():
        y_ref[...] = y_ref[...] + y


@jax.jit
def _forward(x, Wq, Wk, Wv, Wo, Wdq, Wdk):
    x2 = x.reshape(T, C)
    dq, dkT, v = pl.pallas_call(
        _proj_kernel,
        grid=(H,),
        in_specs=[
            pl.BlockSpec((T, C), lambda h: (0, 0)),
            pl.BlockSpec((1, C, D), lambda h: (h, 0, 0)),
            pl.BlockSpec((1, C, D), lambda h: (h, 0, 0)),
            pl.BlockSpec((1, C, D), lambda h: (h, 0, 0)),
            pl.BlockSpec((D, K), lambda h: (0, 0)),
            pl.BlockSpec((D, K), lambda h: (0, 0)),
        ],
        out_specs=(
            pl.BlockSpec((1, T, 2 * K), lambda h: (h, 0, 0)),
            pl.BlockSpec((1, 2 * K, T), lambda h: (h, 0, 0)),
            pl.BlockSpec((1, T, VP), lambda h: (h, 0, 0)),
        ),
        out_shape=(
            jax.ShapeDtypeStruct((H, T, 2 * K), jnp.float32),
            jax.ShapeDtypeStruct((H, 2 * K, T), jnp.float32),
            jax.ShapeDtypeStruct((H, T, VP), jnp.float32),
        ),
        compiler_params=pltpu.CompilerParams(
            dimension_semantics=("arbitrary",),
        ),
    )(x2,
      Wq.T.reshape(C, H, D).transpose(1, 0, 2),
      Wk.T.reshape(C, H, D).transpose(1, 0, 2),
      Wv.T.reshape(C, H, D).transpose(1, 0, 2),
      Wdq.T, Wdk.T)

    nq = T // TQ
    y = pl.pallas_call(
        _attn_kernel,
        grid=(nq, H),
        in_specs=[
            pl.BlockSpec((1, TQ, 2 * K), lambda i, h: (h, i, 0)),
            pl.BlockSpec((1, 2 * K, T), lambda i, h: (h, 0, 0)),
            pl.BlockSpec((1, T, VP), lambda i, h: (h, 0, 0)),
            pl.BlockSpec((D, C), lambda i, h: (h, 0)),
        ],
        out_specs=pl.BlockSpec((TQ, C), lambda i, h: (i, 0)),
        out_shape=jax.ShapeDtypeStruct((T, C), jnp.float32),
        compiler_params=pltpu.CompilerParams(
            dimension_semantics=("arbitrary", "arbitrary"),
        ),
    )(dq, dkT, v, Wo.T)
    return y.reshape(B, T, C)


def kernel(x, Wq, Wk, Wv, Wo, Wdq, Wdk):
    return _forward(x, Wq, Wk, Wv, Wo, Wdq, Wdk)


# speedup vs baseline: 1.1247x; 1.1247x over previous
import functools

import jax
import jax.numpy as jnp
from jax.experimental import pallas as pl
from jax.experimental.pallas import tpu as pltpu

B, T, C = 1, 2048, 768
H, D = 12, 64
K, P = 4, 2
ALPHA, BETA = 2.0, 32.0

TQ = 256
VP = 128


def _proj_kernel(x_ref, wqT_ref, wkT_ref, wvT_ref, wdqT_ref, wdkT_ref,
                 aq_ref, bkT_ref, v_ref):
    x = x_ref[...]
    qh = jnp.dot(x, wqT_ref[0], preferred_element_type=jnp.float32)
    kh = jnp.dot(x, wkT_ref[0], preferred_element_type=jnp.float32)
    scale = jnp.float32(BETA * (P - 1))
    dq = jax.nn.sigmoid(
        jnp.dot(qh, wdqT_ref[...], preferred_element_type=jnp.float32)) * scale
    dk = jax.nn.sigmoid(
        jnp.dot(kh, wdkT_ref[...], preferred_element_type=jnp.float32)) * scale
    aq_ref[0] = dq
    bkT_ref[0] = dk.T
    vh = jnp.dot(x, wvT_ref[0], preferred_element_type=jnp.float32)
    v_ref[0] = jnp.concatenate(
        [vh, jnp.ones((T, 1), jnp.float32), jnp.zeros((T, VP - D - 1), jnp.float32)],
        axis=1)


def _attn_kernel(aq_ref, bkT_ref, v_ref, ind_ref, woT_ref, y_ref):
    i = pl.program_id(0)
    h = pl.program_id(1)
    aq = aq_ref[0]
    abc = jnp.dot(aq, ind_ref[...], preferred_element_type=jnp.float32)
    halfb = jnp.float32(BETA / 2)
    one = jnp.float32(1.0)

    def wblock(j):
        bkT = bkT_ref[0, :, pl.ds(j * TQ, TQ)]
        bcat = jnp.concatenate(
            [bkT[l:l + 1, :] for l in range(K)], axis=1)
        m = jax.nn.sigmoid(halfb - jnp.abs(abc - bcat))
        m0 = m[:, 0 * TQ:1 * TQ]
        m1 = m[:, 1 * TQ:2 * TQ]
        m2 = m[:, 2 * TQ:3 * TQ]
        m3 = m[:, 3 * TQ:4 * TQ]
        lcp = m0 * (one + m1 * (one + m2 * (one + m3)))
        return jnp.exp2(lcp)

    def body(j, acc):
        vblk = v_ref[0, pl.ds(j * TQ, TQ), :]
        w = wblock(j)
        return acc + jnp.dot(w, vblk, preferred_element_type=jnp.float32)

    acc0 = jnp.zeros((TQ, VP), jnp.float32)
    acc = jax.lax.fori_loop(0, i, body, acc0)

    vblk = v_ref[0, pl.ds(i * TQ, TQ), :]
    w = wblock(i)
    rows = jax.lax.broadcasted_iota(jnp.int32, (TQ, TQ), 0)
    cols = jax.lax.broadcasted_iota(jnp.int32, (TQ, TQ), 1)
    w = jnp.where(cols <= rows, w, jnp.float32(0.0))
    acc = acc + jnp.dot(w, vblk, preferred_element_type=jnp.float32)

    out = acc[:, :D] / acc[:, D:D + 1]
    y = jnp.dot(out, woT_ref[...], preferred_element_type=jnp.float32)

    @pl.when(h == 0)
    def _():
        y_ref[...] = y

    @pl.when(h > 0)
    def _():
        y_ref[...] = y_ref[...] + y


@jax.jit
def _forward(x, Wq, Wk, Wv, Wo, Wdq, Wdk):
    x2 = x.reshape(T, C)
    aq, bkT, v = pl.pallas_call(
        _proj_kernel,
        grid=(H,),
        in_specs=[
            pl.BlockSpec((T, C), lambda h: (0, 0)),
            pl.BlockSpec((1, C, D), lambda h: (h, 0, 0)),
            pl.BlockSpec((1, C, D), lambda h: (h, 0, 0)),
            pl.BlockSpec((1, C, D), lambda h: (h, 0, 0)),
            pl.BlockSpec((D, K), lambda h: (0, 0)),
            pl.BlockSpec((D, K), lambda h: (0, 0)),
        ],
        out_specs=(
            pl.BlockSpec((1, T, K), lambda h: (h, 0, 0)),
            pl.BlockSpec((1, K, T), lambda h: (h, 0, 0)),
            pl.BlockSpec((1, T, VP), lambda h: (h, 0, 0)),
        ),
        out_shape=(
            jax.ShapeDtypeStruct((H, T, K), jnp.float32),
            jax.ShapeDtypeStruct((H, K, T), jnp.float32),
            jax.ShapeDtypeStruct((H, T, VP), jnp.float32),
        ),
        compiler_params=pltpu.CompilerParams(
            dimension_semantics=("arbitrary",),
        ),
    )(x2,
      Wq.T.reshape(C, H, D).transpose(1, 0, 2),
      Wk.T.reshape(C, H, D).transpose(1, 0, 2),
      Wv.T.reshape(C, H, D).transpose(1, 0, 2),
      Wdq.T, Wdk.T)

    ind = jnp.repeat(jnp.eye(K, dtype=jnp.float32), TQ, axis=1)

    nq = T // TQ
    y = pl.pallas_call(
        _attn_kernel,
        grid=(nq, H),
        in_specs=[
            pl.BlockSpec((1, TQ, K), lambda i, h: (h, i, 0)),
            pl.BlockSpec((1, K, T), lambda i, h: (h, 0, 0)),
            pl.BlockSpec((1, T, VP), lambda i, h: (h, 0, 0)),
            pl.BlockSpec((K, K * TQ), lambda i, h: (0, 0)),
            pl.BlockSpec((D, C), lambda i, h: (h, 0)),
        ],
        out_specs=pl.BlockSpec((TQ, C), lambda i, h: (i, 0)),
        out_shape=jax.ShapeDtypeStruct((T, C), jnp.float32),
        compiler_params=pltpu.CompilerParams(
            dimension_semantics=("arbitrary", "arbitrary"),
        ),
    )(aq, bkT, v, ind, Wo.T)
    return y.reshape(B, T, C)


def kernel(x, Wq, Wk, Wv, Wo, Wdq, Wdk):
    return _forward(x, Wq, Wk, Wv, Wo, Wdq, Wdk)


# speedup vs baseline: 1.2761x; 1.1346x over previous
import functools

import jax
import jax.numpy as jnp
from jax.experimental import pallas as pl
from jax.experimental.pallas import tpu as pltpu

B, T, C = 1, 2048, 768
H, D = 12, 64
K, P = 4, 2
ALPHA, BETA = 2.0, 32.0

TQ = 256
VP = 128


def _proj_kernel(x_ref, wqT_ref, wkT_ref, wvT_ref, wdqT_ref, wdkT_ref,
                 aq_ref, bkT_ref, v_ref):
    x = x_ref[...]
    qh = jnp.dot(x, wqT_ref[0], preferred_element_type=jnp.float32)
    kh = jnp.dot(x, wkT_ref[0], preferred_element_type=jnp.float32)
    scale = jnp.float32(BETA * (P - 1))
    dq = jax.nn.sigmoid(
        jnp.dot(qh, wdqT_ref[...], preferred_element_type=jnp.float32)) * scale
    dk = jax.nn.sigmoid(
        jnp.dot(kh, wdkT_ref[...], preferred_element_type=jnp.float32)) * scale
    aq_ref[0] = dq
    bkT_ref[0] = dk.T
    vh = jnp.dot(x, wvT_ref[0], preferred_element_type=jnp.float32)
    v_ref[0] = jnp.concatenate(
        [vh, jnp.ones((T, 1), jnp.float32), jnp.zeros((T, VP - D - 1), jnp.float32)],
        axis=1)


def _attn_kernel(aq_ref, bkT_ref, v_ref, ind_ref, mask_ref, woT_ref, y_ref):
    i = pl.program_id(0)
    halfb = jnp.float32(BETA / 2)
    one = jnp.float32(1.0)
    ind = ind_ref[...]
    outs = []
    for h in range(H):
        aq = aq_ref[h]
        abc = jnp.dot(aq, ind, preferred_element_type=jnp.float32)

        def wblock(j, abc=abc, h=h):
            bkT = bkT_ref[h, :, pl.ds(j * TQ, TQ)]
            bcat = jnp.concatenate(
                [bkT[l:l + 1, :] for l in range(K)], axis=1)
            m = jax.nn.sigmoid(halfb - jnp.abs(abc - bcat))
            m0 = m[:, 0 * TQ:1 * TQ]
            m1 = m[:, 1 * TQ:2 * TQ]
            m2 = m[:, 2 * TQ:3 * TQ]
            m3 = m[:, 3 * TQ:4 * TQ]
            lcp = m0 * (one + m1 * (one + m2 * (one + m3)))
            return jnp.exp2(lcp)

        def body(j, acc, h=h, wblock=wblock):
            vblk = v_ref[h, pl.ds(j * TQ, TQ), :]
            w = wblock(j)
            return acc + jnp.dot(w, vblk, preferred_element_type=jnp.float32)

        acc0 = jnp.zeros((TQ, VP), jnp.float32)
        acc = jax.lax.fori_loop(0, i, body, acc0)

        vblk = v_ref[h, pl.ds(i * TQ, TQ), :]
        w = wblock(i) * mask_ref[...]
        acc = acc + jnp.dot(w, vblk, preferred_element_type=jnp.float32)

        outs.append(acc[:, :D] / acc[:, D:D + 1])

    outcat = jnp.concatenate(outs, axis=1)
    y_ref[...] = jnp.dot(outcat, woT_ref[...],
                         preferred_element_type=jnp.float32)


@jax.jit
def _forward(x, Wq, Wk, Wv, Wo, Wdq, Wdk):
    x2 = x.reshape(T, C)
    aq, bkT, v = pl.pallas_call(
        _proj_kernel,
        grid=(H,),
        in_specs=[
            pl.BlockSpec((T, C), lambda h: (0, 0)),
            pl.BlockSpec((1, C, D), lambda h: (h, 0, 0)),
            pl.BlockSpec((1, C, D), lambda h: (h, 0, 0)),
            pl.BlockSpec((1, C, D), lambda h: (h, 0, 0)),
            pl.BlockSpec((D, K), lambda h: (0, 0)),
            pl.BlockSpec((D, K), lambda h: (0, 0)),
        ],
        out_specs=(
            pl.BlockSpec((1, T, K), lambda h: (h, 0, 0)),
            pl.BlockSpec((1, K, T), lambda h: (h, 0, 0)),
            pl.BlockSpec((1, T, VP), lambda h: (h, 0, 0)),
        ),
        out_shape=(
            jax.ShapeDtypeStruct((H, T, K), jnp.float32),
            jax.ShapeDtypeStruct((H, K, T), jnp.float32),
            jax.ShapeDtypeStruct((H, T, VP), jnp.float32),
        ),
        compiler_params=pltpu.CompilerParams(
            dimension_semantics=("arbitrary",),
        ),
    )(x2,
      Wq.T.reshape(C, H, D).transpose(1, 0, 2),
      Wk.T.reshape(C, H, D).transpose(1, 0, 2),
      Wv.T.reshape(C, H, D).transpose(1, 0, 2),
      Wdq.T, Wdk.T)

    ind = jnp.repeat(jnp.eye(K, dtype=jnp.float32), TQ, axis=1)
    mask = jnp.tril(jnp.ones((TQ, TQ), jnp.float32))

    nq = T // TQ
    y = pl.pallas_call(
        _attn_kernel,
        grid=(nq,),
        in_specs=[
            pl.BlockSpec((H, TQ, K), lambda i: (0, i, 0)),
            pl.BlockSpec((H, K, T), lambda i: (0, 0, 0)),
            pl.BlockSpec((H, T, VP), lambda i: (0, 0, 0)),
            pl.BlockSpec((K, K * TQ), lambda i: (0, 0)),
            pl.BlockSpec((TQ, TQ), lambda i: (0, 0)),
            pl.BlockSpec((H * D, C), lambda i: (0, 0)),
        ],
        out_specs=pl.BlockSpec((TQ, C), lambda i: (i, 0)),
        out_shape=jax.ShapeDtypeStruct((T, C), jnp.float32),
        compiler_params=pltpu.CompilerParams(
            dimension_semantics=("arbitrary",),
        ),
    )(aq, bkT, v, ind, mask, Wo.T)
    return y.reshape(B, T, C)


def kernel(x, Wq, Wk, Wv, Wo, Wdq, Wdk):
    return _forward(x, Wq, Wk, Wv, Wo, Wdq, Wdk)


# speedup vs baseline: 1.3386x; 1.0490x over previous
import functools

import jax
import jax.numpy as jnp
from jax.experimental import pallas as pl
from jax.experimental.pallas import tpu as pltpu

B, T, C = 1, 2048, 768
H, D = 12, 64
K, P = 4, 2
ALPHA, BETA = 2.0, 32.0

TQ = 256
VP = 128


def _proj_kernel(x_ref, wqT_ref, wkT_ref, wvT_ref, wdqT_ref, wdkT_ref,
                 aq_ref, bkT_ref, v_ref):
    x = x_ref[...]
    qh = jnp.dot(x, wqT_ref[0], preferred_element_type=jnp.float32)
    kh = jnp.dot(x, wkT_ref[0], preferred_element_type=jnp.float32)
    scale = jnp.float32(BETA * (P - 1))
    dq = jax.nn.sigmoid(
        jnp.dot(qh, wdqT_ref[...], preferred_element_type=jnp.float32)) * scale
    dk = jax.nn.sigmoid(
        jnp.dot(kh, wdkT_ref[...], preferred_element_type=jnp.float32)) * scale
    aq_ref[0] = dq
    bkT_ref[0] = dk.T
    vh = jnp.dot(x, wvT_ref[0], preferred_element_type=jnp.float32)
    v_ref[0] = jnp.concatenate(
        [vh, jnp.ones((T, 1), jnp.float32), jnp.zeros((T, VP - D - 1), jnp.float32)],
        axis=1)


def _attn_kernel(aq_ref, bkT_ref, v_ref, ind_ref, mask_ref, woT_ref, y_ref):
    i = pl.program_id(0)
    halfb = jnp.float32(BETA / 2)
    one = jnp.float32(1.0)
    ind = ind_ref[...]
    outs = []
    for h in range(H):
        aq = aq_ref[h]
        abc = jnp.dot(aq, ind, preferred_element_type=jnp.float32)

        def wblock(j, abc=abc, h=h):
            bkT = bkT_ref[h, :, pl.ds(j * TQ, TQ)]
            bcat = jnp.concatenate(
                [bkT[l:l + 1, :] for l in range(K)], axis=1)
            z = jnp.exp(jnp.abs(abc - bcat) - halfb)
            e0 = one + z[:, 0 * TQ:1 * TQ]
            e1 = one + z[:, 1 * TQ:2 * TQ]
            e2 = one + z[:, 2 * TQ:3 * TQ]
            e3 = one + z[:, 3 * TQ:4 * TQ]
            s1 = e3 * e2
            s0 = s1 * e1
            num = one + e3 + s1 + s0
            den = e0 * s0
            return jnp.exp2(num / den)

        def body(j, acc, h=h, wblock=wblock):
            vblk = v_ref[h, pl.ds(j * TQ, TQ), :]
            w = wblock(j)
            return acc + jnp.dot(w, vblk, preferred_element_type=jnp.float32)

        acc0 = jnp.zeros((TQ, VP), jnp.float32)
        acc = jax.lax.fori_loop(0, i, body, acc0)

        vblk = v_ref[h, pl.ds(i * TQ, TQ), :]
        w = wblock(i) * mask_ref[...]
        acc = acc + jnp.dot(w, vblk, preferred_element_type=jnp.float32)

        outs.append(acc[:, :D] / acc[:, D:D + 1])

    outcat = jnp.concatenate(outs, axis=1)
    y_ref[...] = jnp.dot(outcat, woT_ref[...],
                         preferred_element_type=jnp.float32)


@jax.jit
def _forward(x, Wq, Wk, Wv, Wo, Wdq, Wdk):
    x2 = x.reshape(T, C)
    aq, bkT, v = pl.pallas_call(
        _proj_kernel,
        grid=(H,),
        in_specs=[
            pl.BlockSpec((T, C), lambda h: (0, 0)),
            pl.BlockSpec((1, C, D), lambda h: (h, 0, 0)),
            pl.BlockSpec((1, C, D), lambda h: (h, 0, 0)),
            pl.BlockSpec((1, C, D), lambda h: (h, 0, 0)),
            pl.BlockSpec((D, K), lambda h: (0, 0)),
            pl.BlockSpec((D, K), lambda h: (0, 0)),
        ],
        out_specs=(
            pl.BlockSpec((1, T, K), lambda h: (h, 0, 0)),
            pl.BlockSpec((1, K, T), lambda h: (h, 0, 0)),
            pl.BlockSpec((1, T, VP), lambda h: (h, 0, 0)),
        ),
        out_shape=(
            jax.ShapeDtypeStruct((H, T, K), jnp.float32),
            jax.ShapeDtypeStruct((H, K, T), jnp.float32),
            jax.ShapeDtypeStruct((H, T, VP), jnp.float32),
        ),
        compiler_params=pltpu.CompilerParams(
            dimension_semantics=("arbitrary",),
        ),
    )(x2,
      Wq.T.reshape(C, H, D).transpose(1, 0, 2),
      Wk.T.reshape(C, H, D).transpose(1, 0, 2),
      Wv.T.reshape(C, H, D).transpose(1, 0, 2),
      Wdq.T, Wdk.T)

    ind = jnp.repeat(jnp.eye(K, dtype=jnp.float32), TQ, axis=1)
    mask = jnp.tril(jnp.ones((TQ, TQ), jnp.float32))

    nq = T // TQ
    y = pl.pallas_call(
        _attn_kernel,
        grid=(nq,),
        in_specs=[
            pl.BlockSpec((H, TQ, K), lambda i: (0, i, 0)),
            pl.BlockSpec((H, K, T), lambda i: (0, 0, 0)),
            pl.BlockSpec((H, T, VP), lambda i: (0, 0, 0)),
            pl.BlockSpec((K, K * TQ), lambda i: (0, 0)),
            pl.BlockSpec((TQ, TQ), lambda i: (0, 0)),
            pl.BlockSpec((H * D, C), lambda i: (0, 0)),
        ],
        out_specs=pl.BlockSpec((TQ, C), lambda i: (i, 0)),
        out_shape=jax.ShapeDtypeStruct((T, C), jnp.float32),
        compiler_params=pltpu.CompilerParams(
            dimension_semantics=("arbitrary",),
        ),
    )(aq, bkT, v, ind, mask, Wo.T)
    return y.reshape(B, T, C)


def kernel(x, Wq, Wk, Wv, Wo, Wdq, Wdk):
    return _forward(x, Wq, Wk, Wv, Wo, Wdq, Wdk)


# speedup vs baseline: 1.4106x; 1.0538x over previous
import functools

import jax
import jax.numpy as jnp
from jax.experimental import pallas as pl
from jax.experimental.pallas import tpu as pltpu

B, T, C = 1, 2048, 768
H, D = 12, 64
K, P = 4, 2
ALPHA, BETA = 2.0, 32.0

TQ = 256
VP = 128


LOG2E = 1.4426950408889634


def _proj_kernel(x_ref, wqT_ref, wkT_ref, wvT_ref, wdqT_ref, wdkT_ref,
                 aq_ref, bkT_hi_ref, bkT_lo_ref, v_ref):
    x = x_ref[...]
    qh = jnp.dot(x, wqT_ref[0], preferred_element_type=jnp.float32)
    kh = jnp.dot(x, wkT_ref[0], preferred_element_type=jnp.float32)
    scale = jnp.float32(BETA * LOG2E * (P - 1))
    hb = jnp.float32(BETA * LOG2E / 2)
    dq = jax.nn.sigmoid(
        jnp.dot(qh, wdqT_ref[...], preferred_element_type=jnp.float32)) * scale
    dk = jax.nn.sigmoid(
        jnp.dot(kh, wdkT_ref[...], preferred_element_type=jnp.float32)) * scale
    aq_ref[0] = dq
    dkT = dk.T
    bkT_hi_ref[0] = dkT + hb
    bkT_lo_ref[0] = dkT - hb
    vh = jnp.dot(x, wvT_ref[0], preferred_element_type=jnp.float32)
    v_ref[0] = jnp.concatenate(
        [vh, jnp.ones((T, 1), jnp.float32), jnp.zeros((T, VP - D - 1), jnp.float32)],
        axis=1)


def _attn_kernel(aq_ref, bkT_hi_ref, bkT_lo_ref, v_ref, ind_ref, mask_ref,
                 woT_ref, y_ref):
    i = pl.program_id(0)
    one = jnp.float32(1.0)
    ind = ind_ref[...]
    outs = []
    for h in range(H):
        aq = aq_ref[h]
        abc = jnp.dot(aq, ind, preferred_element_type=jnp.float32)

        def wblock(j, abc=abc, h=h):
            bhi = bkT_hi_ref[h, :, pl.ds(j * TQ, TQ)]
            blo = bkT_lo_ref[h, :, pl.ds(j * TQ, TQ)]
            bcat_hi = jnp.concatenate(
                [bhi[l:l + 1, :] for l in range(K)], axis=1)
            bcat_lo = jnp.concatenate(
                [blo[l:l + 1, :] for l in range(K)], axis=1)
            z = jnp.exp2(jnp.maximum(abc - bcat_hi, bcat_lo - abc))
            e0 = one + z[:, 0 * TQ:1 * TQ]
            e1 = one + z[:, 1 * TQ:2 * TQ]
            e2 = one + z[:, 2 * TQ:3 * TQ]
            e3 = one + z[:, 3 * TQ:4 * TQ]
            s1 = e3 * e2
            s0 = s1 * e1
            num = one + e3 + s1 + s0
            den = e0 * s0
            return jnp.exp2(num * pl.reciprocal(den, approx=True))

        def body(j, acc, h=h, wblock=wblock):
            vblk = v_ref[h, pl.ds(j * TQ, TQ), :]
            w = wblock(j)
            return acc + jnp.dot(w, vblk, preferred_element_type=jnp.float32)

        acc0 = jnp.zeros((TQ, VP), jnp.float32)
        acc = jax.lax.fori_loop(0, i, body, acc0)

        vblk = v_ref[h, pl.ds(i * TQ, TQ), :]
        w = wblock(i) * mask_ref[...]
        acc = acc + jnp.dot(w, vblk, preferred_element_type=jnp.float32)

        outs.append(acc[:, :D] *
                    pl.reciprocal(acc[:, D:D + 1], approx=True))

    outcat = jnp.concatenate(outs, axis=1)
    y_ref[...] = jnp.dot(outcat, woT_ref[...],
                         preferred_element_type=jnp.float32)


@jax.jit
def _forward(x, Wq, Wk, Wv, Wo, Wdq, Wdk):
    x2 = x.reshape(T, C)
    aq, bkT_hi, bkT_lo, v = pl.pallas_call(
        _proj_kernel,
        grid=(H,),
        in_specs=[
            pl.BlockSpec((T, C), lambda h: (0, 0)),
            pl.BlockSpec((1, C, D), lambda h: (h, 0, 0)),
            pl.BlockSpec((1, C, D), lambda h: (h, 0, 0)),
            pl.BlockSpec((1, C, D), lambda h: (h, 0, 0)),
            pl.BlockSpec((D, K), lambda h: (0, 0)),
            pl.BlockSpec((D, K), lambda h: (0, 0)),
        ],
        out_specs=(
            pl.BlockSpec((1, T, K), lambda h: (h, 0, 0)),
            pl.BlockSpec((1, K, T), lambda h: (h, 0, 0)),
            pl.BlockSpec((1, K, T), lambda h: (h, 0, 0)),
            pl.BlockSpec((1, T, VP), lambda h: (h, 0, 0)),
        ),
        out_shape=(
            jax.ShapeDtypeStruct((H, T, K), jnp.float32),
            jax.ShapeDtypeStruct((H, K, T), jnp.float32),
            jax.ShapeDtypeStruct((H, K, T), jnp.float32),
            jax.ShapeDtypeStruct((H, T, VP), jnp.float32),
        ),
        compiler_params=pltpu.CompilerParams(
            dimension_semantics=("arbitrary",),
        ),
    )(x2,
      Wq.T.reshape(C, H, D).transpose(1, 0, 2),
      Wk.T.reshape(C, H, D).transpose(1, 0, 2),
      Wv.T.reshape(C, H, D).transpose(1, 0, 2),
      Wdq.T, Wdk.T)

    ind = jnp.repeat(jnp.eye(K, dtype=jnp.float32), TQ, axis=1)
    mask = jnp.tril(jnp.ones((TQ, TQ), jnp.float32))

    nq = T // TQ
    y = pl.pallas_call(
        _attn_kernel,
        grid=(nq,),
        in_specs=[
            pl.BlockSpec((H, TQ, K), lambda i: (0, i, 0)),
            pl.BlockSpec((H, K, T), lambda i: (0, 0, 0)),
            pl.BlockSpec((H, K, T), lambda i: (0, 0, 0)),
            pl.BlockSpec((H, T, VP), lambda i: (0, 0, 0)),
            pl.BlockSpec((K, K * TQ), lambda i: (0, 0)),
            pl.BlockSpec((TQ, TQ), lambda i: (0, 0)),
            pl.BlockSpec((H * D, C), lambda i: (0, 0)),
        ],
        out_specs=pl.BlockSpec((TQ, C), lambda i: (i, 0)),
        out_shape=jax.ShapeDtypeStruct((T, C), jnp.float32),
        compiler_params=pltpu.CompilerParams(
            dimension_semantics=("arbitrary",),
        ),
    )(aq, bkT_hi, bkT_lo, v, ind, mask, Wo.T)
    return y.reshape(B, T, C)


def kernel(x, Wq, Wk, Wv, Wo, Wdq, Wdk):
    return _forward(x, Wq, Wk, Wv, Wo, Wdq, Wdk)


# speedup vs baseline: 1.4243x; 1.0097x over previous
import functools

import jax
import jax.numpy as jnp
from jax.experimental import pallas as pl
from jax.experimental.pallas import tpu as pltpu

B, T, C = 1, 2048, 768
H, D = 12, 64
K, P = 4, 2
ALPHA, BETA = 2.0, 32.0

TQ = 256
VP = 128


LOG2E = 1.4426950408889634


def _proj_kernel(x_ref, wqT_ref, wkT_ref, wvT_ref, wdqT_ref, wdkT_ref,
                 aq_ref, bkT_hi_ref, bkT_lo_ref, v_ref):
    x = x_ref[...]
    qh = jnp.dot(x, wqT_ref[0], preferred_element_type=jnp.float32)
    kh = jnp.dot(x, wkT_ref[0], preferred_element_type=jnp.float32)
    scale = jnp.float32(BETA * LOG2E * (P - 1))
    hb = jnp.float32(BETA * LOG2E / 2)
    dq = jax.nn.sigmoid(
        jnp.dot(qh, wdqT_ref[...], preferred_element_type=jnp.float32)) * scale
    dk = jax.nn.sigmoid(
        jnp.dot(kh, wdkT_ref[...], preferred_element_type=jnp.float32)) * scale
    aq_ref[0] = dq
    dkT = dk.T
    bkT_hi_ref[0] = dkT + hb
    bkT_lo_ref[0] = dkT - hb
    vh = jnp.dot(x, wvT_ref[0], preferred_element_type=jnp.float32)
    v_ref[0] = jnp.concatenate(
        [vh, jnp.ones((T, 1), jnp.float32), jnp.zeros((T, VP - D - 1), jnp.float32)],
        axis=1)


def _attn_kernel(aq_ref, bkT_hi_ref, bkT_lo_ref, v_ref, ind_ref, mask_ref,
                 woT_ref, y_ref):
    i = pl.program_id(0)
    one = jnp.float32(1.0)
    ind = ind_ref[...]
    outs = []
    for h in range(H):
        aq = aq_ref[h]
        abc = [jnp.broadcast_to(aq[:, l:l + 1], (TQ, TQ)) for l in range(K)]

        def wblock(j, abc=abc, h=h):
            bhi = bkT_hi_ref[h, :, pl.ds(j * TQ, TQ)]
            blo = bkT_lo_ref[h, :, pl.ds(j * TQ, TQ)]
            e = []
            for l in range(K):
                zl = jnp.exp2(jnp.maximum(abc[l] - bhi[l:l + 1, :],
                                          blo[l:l + 1, :] - abc[l]))
                e.append(one + zl)
            s1 = e[3] * e[2]
            s0 = s1 * e[1]
            num = one + e[3] + s1 + s0
            den = e[0] * s0
            return jnp.exp2(num * pl.reciprocal(den, approx=True))

        def body(j, acc, h=h, wblock=wblock):
            vblk = v_ref[h, pl.ds(j * TQ, TQ), :]
            w = wblock(j)
            return acc + jnp.dot(w, vblk, preferred_element_type=jnp.float32)

        acc0 = jnp.zeros((TQ, VP), jnp.float32)
        acc = jax.lax.fori_loop(0, i, body, acc0)

        vblk = v_ref[h, pl.ds(i * TQ, TQ), :]
        w = wblock(i) * mask_ref[...]
        acc = acc + jnp.dot(w, vblk, preferred_element_type=jnp.float32)

        outs.append(acc[:, :D] *
                    pl.reciprocal(acc[:, D:D + 1], approx=True))

    outcat = jnp.concatenate(outs, axis=1)
    y_ref[...] = jnp.dot(outcat, woT_ref[...],
                         preferred_element_type=jnp.float32)


@jax.jit
def _forward(x, Wq, Wk, Wv, Wo, Wdq, Wdk):
    x2 = x.reshape(T, C)
    aq, bkT_hi, bkT_lo, v = pl.pallas_call(
        _proj_kernel,
        grid=(H,),
        in_specs=[
            pl.BlockSpec((T, C), lambda h: (0, 0)),
            pl.BlockSpec((1, C, D), lambda h: (h, 0, 0)),
            pl.BlockSpec((1, C, D), lambda h: (h, 0, 0)),
            pl.BlockSpec((1, C, D), lambda h: (h, 0, 0)),
            pl.BlockSpec((D, K), lambda h: (0, 0)),
            pl.BlockSpec((D, K), lambda h: (0, 0)),
        ],
        out_specs=(
            pl.BlockSpec((1, T, K), lambda h: (h, 0, 0)),
            pl.BlockSpec((1, K, T), lambda h: (h, 0, 0)),
            pl.BlockSpec((1, K, T), lambda h: (h, 0, 0)),
            pl.BlockSpec((1, T, VP), lambda h: (h, 0, 0)),
        ),
        out_shape=(
            jax.ShapeDtypeStruct((H, T, K), jnp.float32),
            jax.ShapeDtypeStruct((H, K, T), jnp.float32),
            jax.ShapeDtypeStruct((H, K, T), jnp.float32),
            jax.ShapeDtypeStruct((H, T, VP), jnp.float32),
        ),
        compiler_params=pltpu.CompilerParams(
            dimension_semantics=("arbitrary",),
        ),
    )(x2,
      Wq.T.reshape(C, H, D).transpose(1, 0, 2),
      Wk.T.reshape(C, H, D).transpose(1, 0, 2),
      Wv.T.reshape(C, H, D).transpose(1, 0, 2),
      Wdq.T, Wdk.T)

    ind = jnp.repeat(jnp.eye(K, dtype=jnp.float32), TQ, axis=1)
    mask = jnp.tril(jnp.ones((TQ, TQ), jnp.float32))

    nq = T // TQ
    y = pl.pallas_call(
        _attn_kernel,
        grid=(nq,),
        in_specs=[
            pl.BlockSpec((H, TQ, K), lambda i: (0, i, 0)),
            pl.BlockSpec((H, K, T), lambda i: (0, 0, 0)),
            pl.BlockSpec((H, K, T), lambda i: (0, 0, 0)),
            pl.BlockSpec((H, T, VP), lambda i: (0, 0, 0)),
            pl.BlockSpec((K, K * TQ), lambda i: (0, 0)),
            pl.BlockSpec((TQ, TQ), lambda i: (0, 0)),
            pl.BlockSpec((H * D, C), lambda i: (0, 0)),
        ],
        out_specs=pl.BlockSpec((TQ, C), lambda i: (i, 0)),
        out_shape=jax.ShapeDtypeStruct((T, C), jnp.float32),
        compiler_params=pltpu.CompilerParams(
            dimension_semantics=("arbitrary",),
        ),
    )(aq, bkT_hi, bkT_lo, v, ind, mask, Wo.T)
    return y.reshape(B, T, C)


def kernel(x, Wq, Wk, Wv, Wo, Wdq, Wdk):
    return _forward(x, Wq, Wk, Wv, Wo, Wdq, Wdk)


# speedup vs baseline: 1.5093x; 1.0597x over previous
import functools

import jax
import jax.numpy as jnp
from jax.experimental import pallas as pl
from jax.experimental.pallas import tpu as pltpu

B, T, C = 1, 2048, 768
H, D = 12, 64
K, P = 4, 2
ALPHA, BETA = 2.0, 32.0

TQ = 256
VP = 128


LOG2E = 1.4426950408889634


def _digit_kernel(x_ref, weq_ref, wek_ref, aq_ref, bkT_hi_ref, bkT_lo_ref):
    x = x_ref[...]
    scale = jnp.float32(BETA * LOG2E * (P - 1))
    hb = jnp.float32(BETA * LOG2E / 2)
    dq = jax.nn.sigmoid(
        jnp.dot(x, weq_ref[...], preferred_element_type=jnp.float32)) * scale
    dk = jax.nn.sigmoid(
        jnp.dot(x, wek_ref[...], preferred_element_type=jnp.float32)) * scale
    for h in range(H):
        aq_ref[h] = dq[:, h * K:(h + 1) * K]
        dkT = dk[:, h * K:(h + 1) * K].T
        bkT_hi_ref[h] = dkT + hb
        bkT_lo_ref[h] = dkT - hb


def _v_kernel(x_ref, wvT_ref, v_ref):
    x = x_ref[...]
    vh = jnp.dot(x, wvT_ref[0], preferred_element_type=jnp.float32)
    v_ref[0] = jnp.concatenate(
        [vh, jnp.ones((T, 1), jnp.float32), jnp.zeros((T, VP - D - 1), jnp.float32)],
        axis=1)


def _attn_kernel(aq_ref, bkT_hi_ref, bkT_lo_ref, v_ref, ind_ref, mask_ref,
                 woT_ref, y_ref):
    i = pl.program_id(0)
    one = jnp.float32(1.0)
    ind = ind_ref[...]
    outs = []
    for h in range(H):
        aq = aq_ref[h]
        abc = [jnp.broadcast_to(aq[:, l:l + 1], (TQ, TQ)) for l in range(K)]

        def wblock(j, abc=abc, h=h):
            bhi = bkT_hi_ref[h, :, pl.ds(j * TQ, TQ)]
            blo = bkT_lo_ref[h, :, pl.ds(j * TQ, TQ)]
            e = []
            for l in range(K):
                zl = jnp.exp2(jnp.maximum(abc[l] - bhi[l:l + 1, :],
                                          blo[l:l + 1, :] - abc[l]))
                e.append(one + zl)
            s1 = e[3] * e[2]
            s0 = s1 * e[1]
            num = one + e[3] + s1 + s0
            den = e[0] * s0
            return jnp.exp2(num * pl.reciprocal(den, approx=True))

        def body(j, acc, h=h, wblock=wblock):
            vblk = v_ref[h, pl.ds(j * TQ, TQ), :]
            w = wblock(j)
            return acc + jnp.dot(w, vblk, preferred_element_type=jnp.float32)

        acc0 = jnp.zeros((TQ, VP), jnp.float32)
        acc = jax.lax.fori_loop(0, i, body, acc0)

        vblk = v_ref[h, pl.ds(i * TQ, TQ), :]
        w = wblock(i) * mask_ref[...]
        acc = acc + jnp.dot(w, vblk, preferred_element_type=jnp.float32)

        outs.append(acc[:, :D] *
                    pl.reciprocal(acc[:, D:D + 1], approx=True))

    outcat = jnp.concatenate(outs, axis=1)
    y_ref[...] = jnp.dot(outcat, woT_ref[...],
                         preferred_element_type=jnp.float32)


@jax.jit
def _forward(x, Wq, Wk, Wv, Wo, Wdq, Wdk):
    x2 = x.reshape(T, C)
    weq = jnp.einsum('chd,kd->chk', Wq.T.reshape(C, H, D), Wdq).reshape(C, H * K)
    wek = jnp.einsum('chd,kd->chk', Wk.T.reshape(C, H, D), Wdk).reshape(C, H * K)

    aq, bkT_hi, bkT_lo = pl.pallas_call(
        _digit_kernel,
        out_shape=(
            jax.ShapeDtypeStruct((H, T, K), jnp.float32),
            jax.ShapeDtypeStruct((H, K, T), jnp.float32),
            jax.ShapeDtypeStruct((H, K, T), jnp.float32),
        ),
    )(x2, weq, wek)

    v = pl.pallas_call(
        _v_kernel,
        grid=(H,),
        in_specs=[
            pl.BlockSpec((T, C), lambda h: (0, 0)),
            pl.BlockSpec((1, C, D), lambda h: (h, 0, 0)),
        ],
        out_specs=pl.BlockSpec((1, T, VP), lambda h: (h, 0, 0)),
        out_shape=jax.ShapeDtypeStruct((H, T, VP), jnp.float32),
        compiler_params=pltpu.CompilerParams(
            dimension_semantics=("arbitrary",),
        ),
    )(x2, Wv.T.reshape(C, H, D).transpose(1, 0, 2))

    ind = jnp.repeat(jnp.eye(K, dtype=jnp.float32), TQ, axis=1)
    mask = jnp.tril(jnp.ones((TQ, TQ), jnp.float32))

    nq = T // TQ
    y = pl.pallas_call(
        _attn_kernel,
        grid=(nq,),
        in_specs=[
            pl.BlockSpec((H, TQ, K), lambda i: (0, i, 0)),
            pl.BlockSpec((H, K, T), lambda i: (0, 0, 0)),
            pl.BlockSpec((H, K, T), lambda i: (0, 0, 0)),
            pl.BlockSpec((H, T, VP), lambda i: (0, 0, 0)),
            pl.BlockSpec((K, K * TQ), lambda i: (0, 0)),
            pl.BlockSpec((TQ, TQ), lambda i: (0, 0)),
            pl.BlockSpec((H * D, C), lambda i: (0, 0)),
        ],
        out_specs=pl.BlockSpec((TQ, C), lambda i: (i, 0)),
        out_shape=jax.ShapeDtypeStruct((T, C), jnp.float32),
        compiler_params=pltpu.CompilerParams(
            dimension_semantics=("arbitrary",),
        ),
    )(aq, bkT_hi, bkT_lo, v, ind, mask, Wo.T)
    return y.reshape(B, T, C)


def kernel(x, Wq, Wk, Wv, Wo, Wdq, Wdk):
    return _forward(x, Wq, Wk, Wv, Wo, Wdq, Wdk)


# speedup vs baseline: 1.5099x; 1.0004x over previous
import jax
import jax.numpy as jnp
from jax.experimental import pallas as pl
from jax.experimental.pallas import tpu as pltpu

B, T, C = 1, 2048, 768
H, D = 12, 64
K, P = 4, 2
ALPHA, BETA = 2.0, 32.0

TQ = 256
VP = 128


LOG2E = 1.4426950408889634


def _digit_kernel(x_ref, weq_ref, wek_ref, aq_ref, bkT_hi_ref, bkT_lo_ref):
    x = x_ref[...]
    scale = jnp.float32(BETA * LOG2E * (P - 1))
    hb = jnp.float32(BETA * LOG2E / 2)
    dq = jax.nn.sigmoid(
        jnp.dot(x, weq_ref[...], preferred_element_type=jnp.float32)) * scale
    dk = jax.nn.sigmoid(
        jnp.dot(x, wek_ref[...], preferred_element_type=jnp.float32)) * scale
    for h in range(H):
        aq_ref[h] = dq[:, h * K:(h + 1) * K]
        dkT = dk[:, h * K:(h + 1) * K].T
        bkT_hi_ref[h] = dkT + hb
        bkT_lo_ref[h] = dkT - hb


def _v_kernel(x_ref, wvT_ref, v_ref):
    x = x_ref[...]
    vh = jnp.dot(x, wvT_ref[0], preferred_element_type=jnp.float32)
    v_ref[0] = jnp.concatenate(
        [vh, jnp.ones((T, 1), jnp.float32), jnp.zeros((T, VP - D - 1), jnp.float32)],
        axis=1)


def _attn_kernel(aq_ref, bkT_hi_ref, bkT_lo_ref, v_ref, mask_ref,
                 woT_ref, y_ref):
    i = pl.program_id(0)
    one = jnp.float32(1.0)
    outs = []
    for h in range(H):
        aq = aq_ref[h]
        abc = [jnp.broadcast_to(aq[:, l:l + 1], (TQ, TQ)) for l in range(K)]

        def wblock(j, abc=abc, h=h):
            bhi = bkT_hi_ref[h, :, pl.ds(j * TQ, TQ)]
            blo = bkT_lo_ref[h, :, pl.ds(j * TQ, TQ)]
            e = []
            for l in range(K):
                zl = jnp.exp2(jnp.maximum(abc[l] - bhi[l:l + 1, :],
                                          blo[l:l + 1, :] - abc[l]))
                e.append(one + zl)
            s1 = e[3] * e[2]
            s0 = s1 * e[1]
            num = one + e[3] + s1 + s0
            den = e[0] * s0
            return jnp.exp2(num * pl.reciprocal(den, approx=True))

        def body(j, acc, h=h, wblock=wblock):
            vblk = v_ref[h, pl.ds(j * TQ, TQ), :]
            w = wblock(j)
            return acc + jnp.dot(w, vblk, preferred_element_type=jnp.float32)

        acc0 = jnp.zeros((TQ, VP), jnp.float32)
        acc = jax.lax.fori_loop(0, i, body, acc0)

        vblk = v_ref[h, pl.ds(i * TQ, TQ), :]
        w = wblock(i) * mask_ref[...]
        acc = acc + jnp.dot(w, vblk, preferred_element_type=jnp.float32)

        outs.append(acc[:, :D] *
                    pl.reciprocal(acc[:, D:D + 1], approx=True))

    outcat = jnp.concatenate(outs, axis=1)
    y_ref[...] = jnp.dot(outcat, woT_ref[...],
                         preferred_element_type=jnp.float32)


@jax.jit
def _forward(x, Wq, Wk, Wv, Wo, Wdq, Wdk):
    x2 = x.reshape(T, C)
    weq = jnp.einsum('chd,kd->chk', Wq.T.reshape(C, H, D), Wdq).reshape(C, H * K)
    wek = jnp.einsum('chd,kd->chk', Wk.T.reshape(C, H, D), Wdk).reshape(C, H * K)

    aq, bkT_hi, bkT_lo = pl.pallas_call(
        _digit_kernel,
        out_shape=(
            jax.ShapeDtypeStruct((H, T, K), jnp.float32),
            jax.ShapeDtypeStruct((H, K, T), jnp.float32),
            jax.ShapeDtypeStruct((H, K, T), jnp.float32),
        ),
    )(x2, weq, wek)

    v = pl.pallas_call(
        _v_kernel,
        grid=(H,),
        in_specs=[
            pl.BlockSpec((T, C), lambda h: (0, 0)),
            pl.BlockSpec((1, C, D), lambda h: (h, 0, 0)),
        ],
        out_specs=pl.BlockSpec((1, T, VP), lambda h: (h, 0, 0)),
        out_shape=jax.ShapeDtypeStruct((H, T, VP), jnp.float32),
        compiler_params=pltpu.CompilerParams(
            dimension_semantics=("arbitrary",),
        ),
    )(x2, Wv.T.reshape(C, H, D).transpose(1, 0, 2))

    mask = jnp.tril(jnp.ones((TQ, TQ), jnp.float32))

    nq = T // TQ
    y = pl.pallas_call(
        _attn_kernel,
        grid=(nq,),
        in_specs=[
            pl.BlockSpec((H, TQ, K), lambda i: (0, i, 0)),
            pl.BlockSpec((H, K, T), lambda i: (0, 0, 0)),
            pl.BlockSpec((H, K, T), lambda i: (0, 0, 0)),
            pl.BlockSpec((H, T, VP), lambda i: (0, 0, 0)),
            pl.BlockSpec((TQ, TQ), lambda i: (0, 0)),
            pl.BlockSpec((H * D, C), lambda i: (0, 0)),
        ],
        out_specs=pl.BlockSpec((TQ, C), lambda i: (i, 0)),
        out_shape=jax.ShapeDtypeStruct((T, C), jnp.float32),
        compiler_params=pltpu.CompilerParams(
            dimension_semantics=("arbitrary",),
        ),
    )(aq, bkT_hi, bkT_lo, v, mask, Wo.T)
    return y.reshape(B, T, C)


def kernel(x, Wq, Wk, Wv, Wo, Wdq, Wdk):
    return _forward(x, Wq, Wk, Wv, Wo, Wdq, Wdk)
